# Initial kernel scaffold; baseline (speedup 1.0000x reference)
import jax
import jax.numpy as jnp
from jax.experimental import pallas as pl


def kernel(x, t, Wt, bt, W0, b0, W1, b1, W2, b2, W3, b3, Wres, Wh1, bh1, Wh2, bh2):
    raise NotImplementedError("write your pallas kernel here")


# speedup vs baseline: 20.0700x; 20.0700x over previous
import functools

import jax
import jax.numpy as jnp
from jax.experimental import pallas as pl
from jax.experimental.pallas import tpu as pltpu

B = 1024
DATA_DIM = 128
COORD = 2
V = DATA_DIM // COORD
HIDDEN = 256
TDIM = 128
N = B * V

BB = 32
R = BB * V


def _silu(v):
    return v * jax.nn.sigmoid(v)


def _stencil_sum(u3):
    nxt = jnp.concatenate([u3[:, 1:], u3[:, :1]], axis=1)
    prv = jnp.concatenate([u3[:, -1:], u3[:, :-1]], axis=1)
    return u3 + nxt + prv


def _body(tf, cpad, Wt, bt, Wc, Wr, Wtp, b0, W1, b1, W2, b2, W3, b3,
          Wh1, bh1, Wh2, bh2, out_ref):
    emb = jnp.sin(tf[...])
    te = _silu(jnp.dot(emb, Wt[...],
                       preferred_element_type=jnp.float32) + bt[...])
    c0 = jnp.dot(te, Wtp[...],
                 preferred_element_type=jnp.float32) + b0[...]

    c = cpad[...]
    ssum = _stencil_sum(c.reshape(BB, V, 8)).reshape(R, 8)
    pre = (jnp.dot(ssum, Wc[...], preferred_element_type=jnp.float32) +
           jnp.dot(c, Wr[...], preferred_element_type=jnp.float32))
    pre = pre.reshape(BB, V, HIDDEN) + c0[:, None, :]
    h = _silu(pre).reshape(R, HIDDEN)

    for W, b in ((W1, b1), (W2, b2), (W3, b3)):
        u = jnp.dot(h, W[...], preferred_element_type=jnp.float32)
        s = _stencil_sum(u.reshape(BB, V, HIDDEN)).reshape(R, HIDDEN)
        h = _silu(s + b[...] + h)

    g = _silu(jnp.dot(h, Wh1[...],
                      preferred_element_type=jnp.float32) + bh1[...])
    out_ref[...] = jnp.dot(g, Wh2[...],
                           preferred_element_type=jnp.float32) + bh2[...]


@jax.jit
def kernel(x, t, Wt, bt, W0, b0, W1, b1, W2, b2, W3, b3, Wres,
           Wh1, bh1, Wh2, bh2):
    f32 = jnp.float32
    half = TDIM // 2
    freqs = jnp.exp(-jnp.log(10000.0) *
                    jnp.arange(half, dtype=f32) / (half - 1))
    freqs2 = jnp.concatenate([freqs, freqs])
    phase = jnp.concatenate([jnp.zeros((half,), f32),
                             jnp.full((half,), jnp.pi / 2, f32)])
    tf = t.astype(f32)[:, None] * freqs2[None, :] + phase[None, :]

    coords = x.reshape(N, COORD)
    cpad = jnp.pad(coords, ((0, 0), (0, 8 - COORD)))

    third = jnp.float32(1.0 / 3.0)
    Wc = jnp.pad(W0[:COORD] * third, ((0, 8 - COORD), (0, 0)))
    Wr = jnp.pad(Wres[:COORD], ((0, 8 - COORD), (0, 0)))
    Wtp = W0[COORD:] + Wres[COORD:]
    Wh2p = jnp.pad(Wh2, ((0, 0), (0, 8 - COORD)))
    bh2p = jnp.pad(bh2, (0, 8 - COORD)).reshape(1, 8)

    row = lambda v: v.reshape(1, -1)
    grid = B // BB
    rep = lambda i: (0, 0)

    out = pl.pallas_call(
        _body,
        grid=(grid,),
        in_specs=[
            pl.BlockSpec((BB, TDIM), lambda i: (i, 0)),
            pl.BlockSpec((R, 8), lambda i: (i, 0)),
            pl.BlockSpec((TDIM, TDIM), rep),
            pl.BlockSpec((1, TDIM), rep),
            pl.BlockSpec((8, HIDDEN), rep),
            pl.BlockSpec((8, HIDDEN), rep),
            pl.BlockSpec((TDIM, HIDDEN), rep),
            pl.BlockSpec((1, HIDDEN), rep),
            pl.BlockSpec((HIDDEN, HIDDEN), rep),
            pl.BlockSpec((1, HIDDEN), rep),
            pl.BlockSpec((HIDDEN, HIDDEN), rep),
            pl.BlockSpec((1, HIDDEN), rep),
            pl.BlockSpec((HIDDEN, HIDDEN), rep),
            pl.BlockSpec((1, HIDDEN), rep),
            pl.BlockSpec((HIDDEN, HIDDEN), rep),
            pl.BlockSpec((1, HIDDEN), rep),
            pl.BlockSpec((HIDDEN, 8), rep),
            pl.BlockSpec((1, 8), rep),
        ],
        out_specs=pl.BlockSpec((R, 8), lambda i: (i, 0)),
        out_shape=jax.ShapeDtypeStruct((N, 8), f32),
        compiler_params=pltpu.CompilerParams(
            dimension_semantics=("parallel",)),
    )(tf, cpad, Wt, row(bt), Wc, Wr, Wtp, row(b0),
      W1 * third, row(b1), W2 * third, row(b2), W3 * third, row(b3),
      Wh1, row(bh1), Wh2p, bh2p)

    return out[:, :COORD].reshape(B, DATA_DIM)


# speedup vs baseline: 20.4394x; 1.0184x over previous
import functools

import jax
import jax.numpy as jnp
from jax.experimental import pallas as pl
from jax.experimental.pallas import tpu as pltpu

B = 1024
DATA_DIM = 128
COORD = 2
V = DATA_DIM // COORD
HIDDEN = 256
TDIM = 128
N = B * V

BB = 64
R = BB * V


def _silu(v):
    return v * jax.nn.sigmoid(v)


def _stencil_sum(u3):
    nxt = jnp.concatenate([u3[:, 1:], u3[:, :1]], axis=1)
    prv = jnp.concatenate([u3[:, -1:], u3[:, :-1]], axis=1)
    return u3 + nxt + prv


def _body(tf, cpad, Wt, bt, Wc, Wr, Wtp, b0, W1, b1, W2, b2, W3, b3,
          Wh1, bh1, Wh2, bh2, out_ref):
    emb = jnp.sin(tf[...])
    te = _silu(jnp.dot(emb, Wt[...],
                       preferred_element_type=jnp.float32) + bt[...])
    c0 = jnp.dot(te, Wtp[...],
                 preferred_element_type=jnp.float32) + b0[...]

    c = cpad[...]
    ssum = _stencil_sum(c.reshape(BB, V, 8)).reshape(R, 8)
    pre = (jnp.dot(ssum, Wc[...], preferred_element_type=jnp.float32) +
           jnp.dot(c, Wr[...], preferred_element_type=jnp.float32))
    pre = pre.reshape(BB, V, HIDDEN) + c0[:, None, :]
    h = _silu(pre).reshape(R, HIDDEN)

    for W, b in ((W1, b1), (W2, b2), (W3, b3)):
        u = jnp.dot(h, W[...], preferred_element_type=jnp.float32)
        s = _stencil_sum(u.reshape(BB, V, HIDDEN)).reshape(R, HIDDEN)
        h = _silu(s + b[...] + h)

    g = _silu(jnp.dot(h, Wh1[...],
                      preferred_element_type=jnp.float32) + bh1[...])
    out_ref[...] = jnp.dot(g, Wh2[...],
                           preferred_element_type=jnp.float32) + bh2[...]


@jax.jit
def kernel(x, t, Wt, bt, W0, b0, W1, b1, W2, b2, W3, b3, Wres,
           Wh1, bh1, Wh2, bh2):
    f32 = jnp.float32
    half = TDIM // 2
    freqs = jnp.exp(-jnp.log(10000.0) *
                    jnp.arange(half, dtype=f32) / (half - 1))
    freqs2 = jnp.concatenate([freqs, freqs])
    phase = jnp.concatenate([jnp.zeros((half,), f32),
                             jnp.full((half,), jnp.pi / 2, f32)])
    tf = t.astype(f32)[:, None] * freqs2[None, :] + phase[None, :]

    coords = x.reshape(N, COORD)
    cpad = jnp.pad(coords, ((0, 0), (0, 8 - COORD)))

    third = jnp.float32(1.0 / 3.0)
    Wc = jnp.pad(W0[:COORD] * third, ((0, 8 - COORD), (0, 0)))
    Wr = jnp.pad(Wres[:COORD], ((0, 8 - COORD), (0, 0)))
    Wtp = W0[COORD:] + Wres[COORD:]
    Wh2p = jnp.pad(Wh2, ((0, 0), (0, 8 - COORD)))
    bh2p = jnp.pad(bh2, (0, 8 - COORD)).reshape(1, 8)

    row = lambda v: v.reshape(1, -1)
    grid = B // BB
    rep = lambda i: (0, 0)

    out = pl.pallas_call(
        _body,
        grid=(grid,),
        in_specs=[
            pl.BlockSpec((BB, TDIM), lambda i: (i, 0)),
            pl.BlockSpec((R, 8), lambda i: (i, 0)),
            pl.BlockSpec((TDIM, TDIM), rep),
            pl.BlockSpec((1, TDIM), rep),
            pl.BlockSpec((8, HIDDEN), rep),
            pl.BlockSpec((8, HIDDEN), rep),
            pl.BlockSpec((TDIM, HIDDEN), rep),
            pl.BlockSpec((1, HIDDEN), rep),
            pl.BlockSpec((HIDDEN, HIDDEN), rep),
            pl.BlockSpec((1, HIDDEN), rep),
            pl.BlockSpec((HIDDEN, HIDDEN), rep),
            pl.BlockSpec((1, HIDDEN), rep),
            pl.BlockSpec((HIDDEN, HIDDEN), rep),
            pl.BlockSpec((1, HIDDEN), rep),
            pl.BlockSpec((HIDDEN, HIDDEN), rep),
            pl.BlockSpec((1, HIDDEN), rep),
            pl.BlockSpec((HIDDEN, 8), rep),
            pl.BlockSpec((1, 8), rep),
        ],
        out_specs=pl.BlockSpec((R, 8), lambda i: (i, 0)),
        out_shape=jax.ShapeDtypeStruct((N, 8), f32),
        compiler_params=pltpu.CompilerParams(
            dimension_semantics=("parallel",)),
    )(tf, cpad, Wt, row(bt), Wc, Wr, Wtp, row(b0),
      W1 * third, row(b1), W2 * third, row(b2), W3 * third, row(b3),
      Wh1, row(bh1), Wh2p, bh2p)

    return out[:, :COORD].reshape(B, DATA_DIM)


# speedup vs baseline: 22.4047x; 1.0962x over previous
import functools

import jax
import jax.numpy as jnp
from jax.experimental import pallas as pl
from jax.experimental.pallas import tpu as pltpu

B = 1024
DATA_DIM = 128
COORD = 2
V = DATA_DIM // COORD
HIDDEN = 256
TDIM = 128
N = B * V

BB = 64
R = BB * V


def _silu(v):
    return v * jax.nn.sigmoid(v)


def _stencil_sum(u3, A):
    Ab = jnp.broadcast_to(A[None], (BB, V, V))
    return jax.lax.dot_general(Ab, u3, (((2,), (1,)), ((0,), (0,))),
                               preferred_element_type=jnp.float32)


def _body(tf, cpad, A, Wt, bt, Wc, Wr, Wtp, b0, W1, b1, W2, b2, W3, b3,
          Wh1, bh1, Wh2, bh2, out_ref):
    emb = jnp.sin(tf[...])
    te = _silu(jnp.dot(emb, Wt[...],
                       preferred_element_type=jnp.float32) + bt[...])
    c0 = jnp.dot(te, Wtp[...],
                 preferred_element_type=jnp.float32) + b0[...]

    Av = A[...]
    c = cpad[...]
    ssum = _stencil_sum(c.reshape(BB, V, 8), Av).reshape(R, 8)
    pre = (jnp.dot(ssum, Wc[...], preferred_element_type=jnp.float32) +
           jnp.dot(c, Wr[...], preferred_element_type=jnp.float32))
    pre = pre.reshape(BB, V, HIDDEN) + c0[:, None, :]
    h = _silu(pre).reshape(R, HIDDEN)

    for W, b in ((W1, b1), (W2, b2), (W3, b3)):
        u = jnp.dot(h, W[...], preferred_element_type=jnp.float32)
        s = _stencil_sum(u.reshape(BB, V, HIDDEN), Av).reshape(R, HIDDEN)
        h = _silu(s + b[...] + h)

    g = _silu(jnp.dot(h, Wh1[...],
                      preferred_element_type=jnp.float32) + bh1[...])
    out_ref[...] = jnp.dot(g, Wh2[...],
                           preferred_element_type=jnp.float32) + bh2[...]


@jax.jit
def kernel(x, t, Wt, bt, W0, b0, W1, b1, W2, b2, W3, b3, Wres,
           Wh1, bh1, Wh2, bh2):
    f32 = jnp.float32
    half = TDIM // 2
    freqs = jnp.exp(-jnp.log(10000.0) *
                    jnp.arange(half, dtype=f32) / (half - 1))
    freqs2 = jnp.concatenate([freqs, freqs])
    phase = jnp.concatenate([jnp.zeros((half,), f32),
                             jnp.full((half,), jnp.pi / 2, f32)])
    tf = t.astype(f32)[:, None] * freqs2[None, :] + phase[None, :]

    coords = x.reshape(N, COORD)
    cpad = jnp.pad(coords, ((0, 0), (0, 8 - COORD)))

    vi = jnp.arange(V, dtype=jnp.int32)
    dd = jnp.abs(vi[:, None] - vi[None, :])
    Amat = ((dd == 0) | (dd == 1) | (dd == V - 1)).astype(f32)

    third = jnp.float32(1.0 / 3.0)
    Wc = jnp.pad(W0[:COORD] * third, ((0, 8 - COORD), (0, 0)))
    Wr = jnp.pad(Wres[:COORD], ((0, 8 - COORD), (0, 0)))
    Wtp = W0[COORD:] + Wres[COORD:]
    Wh2p = jnp.pad(Wh2, ((0, 0), (0, 8 - COORD)))
    bh2p = jnp.pad(bh2, (0, 8 - COORD)).reshape(1, 8)

    row = lambda v: v.reshape(1, -1)
    grid = B // BB
    rep = lambda i: (0, 0)

    out = pl.pallas_call(
        _body,
        grid=(grid,),
        in_specs=[
            pl.BlockSpec((BB, TDIM), lambda i: (i, 0)),
            pl.BlockSpec((R, 8), lambda i: (i, 0)),
            pl.BlockSpec((V, V), rep),
            pl.BlockSpec((TDIM, TDIM), rep),
            pl.BlockSpec((1, TDIM), rep),
            pl.BlockSpec((8, HIDDEN), rep),
            pl.BlockSpec((8, HIDDEN), rep),
            pl.BlockSpec((TDIM, HIDDEN), rep),
            pl.BlockSpec((1, HIDDEN), rep),
            pl.BlockSpec((HIDDEN, HIDDEN), rep),
            pl.BlockSpec((1, HIDDEN), rep),
            pl.BlockSpec((HIDDEN, HIDDEN), rep),
            pl.BlockSpec((1, HIDDEN), rep),
            pl.BlockSpec((HIDDEN, HIDDEN), rep),
            pl.BlockSpec((1, HIDDEN), rep),
            pl.BlockSpec((HIDDEN, HIDDEN), rep),
            pl.BlockSpec((1, HIDDEN), rep),
            pl.BlockSpec((HIDDEN, 8), rep),
            pl.BlockSpec((1, 8), rep),
        ],
        out_specs=pl.BlockSpec((R, 8), lambda i: (i, 0)),
        out_shape=jax.ShapeDtypeStruct((N, 8), f32),
        compiler_params=pltpu.CompilerParams(
            dimension_semantics=("parallel",)),
    )(tf, cpad, Amat, Wt, row(bt), Wc, Wr, Wtp, row(b0),
      W1 * third, row(b1), W2 * third, row(b2), W3 * third, row(b3),
      Wh1, row(bh1), Wh2p, bh2p)

    return out[:, :COORD].reshape(B, DATA_DIM)


# speedup vs baseline: 22.7000x; 1.0132x over previous
import functools

import jax
import jax.numpy as jnp
from jax.experimental import pallas as pl
from jax.experimental.pallas import tpu as pltpu

B = 1024
DATA_DIM = 128
COORD = 2
V = DATA_DIM // COORD
HIDDEN = 256
TDIM = 128
N = B * V

BB = 64
R = BB * V


def _silu(v):
    return 0.5 * v * (1.0 + jnp.tanh(0.5 * v))


def _stencil_sum(u3, A):
    Ab = jnp.broadcast_to(A[None], (BB, V, V))
    return jax.lax.dot_general(Ab, u3, (((2,), (1,)), ((0,), (0,))),
                               preferred_element_type=jnp.float32)


def _body(tf, cpad, A, Wt, bt, Wc, Wr, Wtp, b0, W1, b1, W2, b2, W3, b3,
          Wh1, bh1, Wh2, bh2, out_ref):
    emb = jnp.sin(tf[...])
    te = _silu(jnp.dot(emb, Wt[...],
                       preferred_element_type=jnp.float32) + bt[...])
    c0 = jnp.dot(te, Wtp[...],
                 preferred_element_type=jnp.float32) + b0[...]

    Av = A[...]
    c = cpad[...]
    ssum = _stencil_sum(c.reshape(BB, V, 8), Av).reshape(R, 8)
    pre = (jnp.dot(ssum, Wc[...], preferred_element_type=jnp.float32) +
           jnp.dot(c, Wr[...], preferred_element_type=jnp.float32))
    pre = pre.reshape(BB, V, HIDDEN) + c0[:, None, :]
    h = _silu(pre).reshape(R, HIDDEN)

    for W, b in ((W1, b1), (W2, b2), (W3, b3)):
        u = jnp.dot(h, W[...], preferred_element_type=jnp.float32)
        s = _stencil_sum(u.reshape(BB, V, HIDDEN), Av).reshape(R, HIDDEN)
        h = _silu(s + b[...] + h)

    g = _silu(jnp.dot(h, Wh1[...],
                      preferred_element_type=jnp.float32) + bh1[...])
    out_ref[...] = jnp.dot(g, Wh2[...],
                           preferred_element_type=jnp.float32) + bh2[...]


@jax.jit
def kernel(x, t, Wt, bt, W0, b0, W1, b1, W2, b2, W3, b3, Wres,
           Wh1, bh1, Wh2, bh2):
    f32 = jnp.float32
    half = TDIM // 2
    freqs = jnp.exp(-jnp.log(10000.0) *
                    jnp.arange(half, dtype=f32) / (half - 1))
    freqs2 = jnp.concatenate([freqs, freqs])
    phase = jnp.concatenate([jnp.zeros((half,), f32),
                             jnp.full((half,), jnp.pi / 2, f32)])
    tf = t.astype(f32)[:, None] * freqs2[None, :] + phase[None, :]

    coords = x.reshape(N, COORD)
    cpad = jnp.pad(coords, ((0, 0), (0, 8 - COORD)))

    vi = jnp.arange(V, dtype=jnp.int32)
    dd = jnp.abs(vi[:, None] - vi[None, :])
    Amat = ((dd == 0) | (dd == 1) | (dd == V - 1)).astype(f32)

    third = jnp.float32(1.0 / 3.0)
    Wc = jnp.pad(W0[:COORD] * third, ((0, 8 - COORD), (0, 0)))
    Wr = jnp.pad(Wres[:COORD], ((0, 8 - COORD), (0, 0)))
    Wtp = W0[COORD:] + Wres[COORD:]
    Wh2p = jnp.pad(Wh2, ((0, 0), (0, 8 - COORD)))
    bh2p = jnp.pad(bh2, (0, 8 - COORD)).reshape(1, 8)

    row = lambda v: v.reshape(1, -1)
    grid = B // BB
    rep = lambda i: (0, 0)

    out = pl.pallas_call(
        _body,
        grid=(grid,),
        in_specs=[
            pl.BlockSpec((BB, TDIM), lambda i: (i, 0)),
            pl.BlockSpec((R, 8), lambda i: (i, 0)),
            pl.BlockSpec((V, V), rep),
            pl.BlockSpec((TDIM, TDIM), rep),
            pl.BlockSpec((1, TDIM), rep),
            pl.BlockSpec((8, HIDDEN), rep),
            pl.BlockSpec((8, HIDDEN), rep),
            pl.BlockSpec((TDIM, HIDDEN), rep),
            pl.BlockSpec((1, HIDDEN), rep),
            pl.BlockSpec((HIDDEN, HIDDEN), rep),
            pl.BlockSpec((1, HIDDEN), rep),
            pl.BlockSpec((HIDDEN, HIDDEN), rep),
            pl.BlockSpec((1, HIDDEN), rep),
            pl.BlockSpec((HIDDEN, HIDDEN), rep),
            pl.BlockSpec((1, HIDDEN), rep),
            pl.BlockSpec((HIDDEN, HIDDEN), rep),
            pl.BlockSpec((1, HIDDEN), rep),
            pl.BlockSpec((HIDDEN, 8), rep),
            pl.BlockSpec((1, 8), rep),
        ],
        out_specs=pl.BlockSpec((R, 8), lambda i: (i, 0)),
        out_shape=jax.ShapeDtypeStruct((N, 8), f32),
        compiler_params=pltpu.CompilerParams(
            dimension_semantics=("parallel",)),
    )(tf, cpad, Amat, Wt, row(bt), Wc, Wr, Wtp, row(b0),
      W1 * third, row(b1), W2 * third, row(b2), W3 * third, row(b3),
      Wh1, row(bh1), Wh2p, bh2p)

    return out[:, :COORD].reshape(B, DATA_DIM)


# speedup vs baseline: 26.5270x; 1.1686x over previous
import jax
import jax.numpy as jnp
from jax.experimental import pallas as pl
from jax.experimental.pallas import tpu as pltpu

B = 1024
DATA_DIM = 128
COORD = 2
V = DATA_DIM // COORD
HIDDEN = 256
TDIM = 128
N = B * V

BB = 64
R = BB * V


def _silu(v):
    return 0.5 * v * (1.0 + jnp.tanh(0.5 * v))


def _spmm(u3, A3):
    Ab = jnp.broadcast_to(A3[None], (BB, V, V))
    return jax.lax.dot_general(Ab, u3, (((2,), (1,)), ((0,), (0,))),
                               preferred_element_type=jnp.float32)


def _body(coords, tcol, Wt, bt, W0, b0, W1, b1, W2, b2, W3, b3, Wres,
          Wh1, bh1, Wh2, bh2, out_ref):
    f32 = jnp.float32
    dot = lambda a, b: jnp.dot(a, b, preferred_element_type=f32)

    ri = jax.lax.broadcasted_iota(jnp.int32, (V, V), 0)
    ci = jax.lax.broadcasted_iota(jnp.int32, (V, V), 1)
    dd = jnp.abs(ri - ci)
    A3 = jnp.where((dd == 0) | (dd == 1) | (dd == V - 1),
                   f32(1.0 / 3.0), f32(0.0))

    li = jax.lax.broadcasted_iota(jnp.int32, (1, TDIM), 1)
    lm = jnp.where(li >= TDIM // 2, li - TDIM // 2, li).astype(f32)
    freqs = jnp.exp(f32(-jnp.log(10000.0) / (TDIM // 2 - 1)) * lm)
    phase = jnp.where(li >= TDIM // 2, f32(jnp.pi / 2), f32(0.0))
    tf = tcol[...].astype(f32) * freqs + phase

    te = _silu(dot(jnp.sin(tf), Wt[...]) + bt[...])
    Wtp = W0[COORD:, :] + Wres[COORD:, :]
    c0 = dot(te, Wtp) + b0[...]

    c = coords[...]
    sc = _spmm(c.reshape(BB, V, COORD), A3).reshape(R, COORD)
    pre = dot(sc, W0[:COORD, :]) + dot(c, Wres[:COORD, :])
    h = _silu(pre.reshape(BB, V, HIDDEN) + c0[:, None, :]).reshape(R, HIDDEN)

    for W, b in ((W1, b1), (W2, b2), (W3, b3)):
        u = dot(h, W[...])
        s = _spmm(u.reshape(BB, V, HIDDEN), A3).reshape(R, HIDDEN)
        h = _silu(s + b[...] + h)

    g = _silu(dot(h, Wh1[...]) + bh1[...])
    out_ref[...] = dot(g, Wh2[...]) + bh2[...]


@jax.jit
def kernel(x, t, Wt, bt, W0, b0, W1, b1, W2, b2, W3, b3, Wres,
           Wh1, bh1, Wh2, bh2):
    grid = B // BB
    rep = lambda i: (0, 0)
    row = lambda v: v.reshape(1, -1)

    out = pl.pallas_call(
        _body,
        grid=(grid,),
        in_specs=[
            pl.BlockSpec((R, COORD), lambda i: (i, 0)),
            pl.BlockSpec((BB, 1), lambda i: (i, 0)),
            pl.BlockSpec((TDIM, TDIM), rep),
            pl.BlockSpec((1, TDIM), rep),
            pl.BlockSpec((COORD + TDIM, HIDDEN), rep),
            pl.BlockSpec((1, HIDDEN), rep),
            pl.BlockSpec((HIDDEN, HIDDEN), rep),
            pl.BlockSpec((1, HIDDEN), rep),
            pl.BlockSpec((HIDDEN, HIDDEN), rep),
            pl.BlockSpec((1, HIDDEN), rep),
            pl.BlockSpec((HIDDEN, HIDDEN), rep),
            pl.BlockSpec((1, HIDDEN), rep),
            pl.BlockSpec((COORD + TDIM, HIDDEN), rep),
            pl.BlockSpec((HIDDEN, HIDDEN), rep),
            pl.BlockSpec((1, HIDDEN), rep),
            pl.BlockSpec((HIDDEN, COORD), rep),
            pl.BlockSpec((1, COORD), rep),
        ],
        out_specs=pl.BlockSpec((R, COORD), lambda i: (i, 0)),
        out_shape=jax.ShapeDtypeStruct((N, COORD), jnp.float32),
        compiler_params=pltpu.CompilerParams(
            dimension_semantics=("parallel",)),
    )(x.reshape(N, COORD), t.reshape(B, 1), Wt, row(bt), W0, row(b0),
      W1, row(b1), W2, row(b2), W3, row(b3), Wres,
      Wh1, row(bh1), Wh2, row(bh2))

    return out.reshape(B, DATA_DIM)


# speedup vs baseline: 27.1537x; 1.0236x over previous
import jax
import jax.numpy as jnp
from jax.experimental import pallas as pl
from jax.experimental.pallas import tpu as pltpu

B = 1024
DATA_DIM = 128
COORD = 2
V = DATA_DIM // COORD
HIDDEN = 256
TDIM = 128
N = B * V

BB = 128
R = BB * V


def _silu(v):
    return 0.5 * v * (1.0 + jnp.tanh(0.5 * v))


def _spmm(u3, A3):
    Ab = jnp.broadcast_to(A3[None], (BB, V, V))
    return jax.lax.dot_general(Ab, u3, (((2,), (1,)), ((0,), (0,))),
                               preferred_element_type=jnp.float32)


def _body(coords, tcol, Wt, bt, W0, b0, W1, b1, W2, b2, W3, b3, Wres,
          Wh1, bh1, Wh2, bh2, out_ref):
    f32 = jnp.float32
    dot = lambda a, b: jnp.dot(a, b, preferred_element_type=f32)

    ri = jax.lax.broadcasted_iota(jnp.int32, (V, V), 0)
    ci = jax.lax.broadcasted_iota(jnp.int32, (V, V), 1)
    dd = jnp.abs(ri - ci)
    A3 = jnp.where((dd == 0) | (dd == 1) | (dd == V - 1),
                   f32(1.0 / 3.0), f32(0.0))

    li = jax.lax.broadcasted_iota(jnp.int32, (1, TDIM), 1)
    lm = jnp.where(li >= TDIM // 2, li - TDIM // 2, li).astype(f32)
    freqs = jnp.exp(f32(-jnp.log(10000.0) / (TDIM // 2 - 1)) * lm)
    phase = jnp.where(li >= TDIM // 2, f32(jnp.pi / 2), f32(0.0))
    tf = tcol[...].astype(f32) * freqs + phase

    te = _silu(dot(jnp.sin(tf), Wt[...]) + bt[...])
    Wtp = W0[COORD:, :] + Wres[COORD:, :]
    c0 = dot(te, Wtp) + b0[...]

    c = coords[...]
    sc = _spmm(c.reshape(BB, V, COORD), A3).reshape(R, COORD)
    pre = dot(sc, W0[:COORD, :]) + dot(c, Wres[:COORD, :])
    h = _silu(pre.reshape(BB, V, HIDDEN) + c0[:, None, :]).reshape(R, HIDDEN)

    for W, b in ((W1, b1), (W2, b2), (W3, b3)):
        u = dot(h, W[...])
        s = _spmm(u.reshape(BB, V, HIDDEN), A3).reshape(R, HIDDEN)
        h = _silu(s + b[...] + h)

    g = _silu(dot(h, Wh1[...]) + bh1[...])
    out_ref[...] = dot(g, Wh2[...]) + bh2[...]


@jax.jit
def kernel(x, t, Wt, bt, W0, b0, W1, b1, W2, b2, W3, b3, Wres,
           Wh1, bh1, Wh2, bh2):
    grid = B // BB
    rep = lambda i: (0, 0)
    row = lambda v: v.reshape(1, -1)

    out = pl.pallas_call(
        _body,
        grid=(grid,),
        in_specs=[
            pl.BlockSpec((R, COORD), lambda i: (i, 0)),
            pl.BlockSpec((BB, 1), lambda i: (i, 0)),
            pl.BlockSpec((TDIM, TDIM), rep),
            pl.BlockSpec((1, TDIM), rep),
            pl.BlockSpec((COORD + TDIM, HIDDEN), rep),
            pl.BlockSpec((1, HIDDEN), rep),
            pl.BlockSpec((HIDDEN, HIDDEN), rep),
            pl.BlockSpec((1, HIDDEN), rep),
            pl.BlockSpec((HIDDEN, HIDDEN), rep),
            pl.BlockSpec((1, HIDDEN), rep),
            pl.BlockSpec((HIDDEN, HIDDEN), rep),
            pl.BlockSpec((1, HIDDEN), rep),
            pl.BlockSpec((COORD + TDIM, HIDDEN), rep),
            pl.BlockSpec((HIDDEN, HIDDEN), rep),
            pl.BlockSpec((1, HIDDEN), rep),
            pl.BlockSpec((HIDDEN, COORD), rep),
            pl.BlockSpec((1, COORD), rep),
        ],
        out_specs=pl.BlockSpec((R, COORD), lambda i: (i, 0)),
        out_shape=jax.ShapeDtypeStruct((N, COORD), jnp.float32),
        compiler_params=pltpu.CompilerParams(
            dimension_semantics=("parallel",)),
    )(x.reshape(N, COORD), t.reshape(B, 1), Wt, row(bt), W0, row(b0),
      W1, row(b1), W2, row(b2), W3, row(b3), Wres,
      Wh1, row(bh1), Wh2, row(bh2))

    return out.reshape(B, DATA_DIM)
